# baseline (speedup 1.0000x reference)
import jax
import jax.numpy as jnp
from jax import lax
from jax.experimental import pallas as pl
from jax.experimental.pallas import tpu as pltpu
from jax.experimental.pallas import tpu_sc as plsc

NC = 2
NS = 16
CHUNK = 128
NSLOT = 8
MB = 256


def _sc_edge_scatter(table, src2d, dst2d, npad, d, ch_per_worker):
  rows_per_sub = npad // NS
  n_row_blk = rows_per_sub // CHUNK
  nz = CHUNK * (d // 16)
  assert ch_per_worker % NSLOT == 0
  ngrp = ch_per_worker // NSLOT

  def body(table_hbm, src_hbm, dst_hbm, out_hbm, idx_s, idx_d, *rest):
    rows = list(rest[:NSLOT])
    accum = rest[NSLOT]
    gsem = list(rest[NSLOT + 1:2 * NSLOT + 1])
    ssem = list(rest[2 * NSLOT + 1:3 * NSLOT + 1])
    cid = lax.axis_index("c")
    sid = lax.axis_index("s")
    w = sid * NC + cid
    base_r = sid * rows_per_sub

    def zr(i, _):
      rows[0][i // (d // 16), pl.ds((i % (d // 16)) * 16, 16)] = jnp.zeros(
          (16,), jnp.float32)
      return 0
    lax.fori_loop(0, nz, zr, 0)

    def zb(k, _):
      pltpu.sync_copy(rows[0], accum.at[pl.ds(base_r + k * CHUNK, CHUNK)])
      return 0
    lax.fori_loop(0, n_row_blk, zb, 0)

    pltpu.sync_copy(src_hbm.at[pl.ds(w * ch_per_worker, ch_per_worker)], idx_s)
    pltpu.sync_copy(dst_hbm.at[pl.ds(w * ch_per_worker, ch_per_worker)], idx_d)
    plsc.subcore_barrier()

    def fire_g(j, b):
      pltpu.async_copy(table_hbm.at[idx_s.at[j]], rows[b], gsem[b])

    def wait_g(j, b):
      pltpu.make_async_copy(table_hbm.at[idx_s.at[j]], rows[b],
                            gsem[b]).wait()

    def fire_s(j, b):
      pltpu.async_copy(rows[b], accum.at[idx_d.at[j]], ssem[b], add=True)

    def wait_s(j, b):
      pltpu.make_async_copy(rows[b], accum.at[idx_d.at[j]], ssem[b]).wait()

    for b in range(NSLOT):
      fire_g(b, b)

    def grp(g, _):
      j0 = g * NSLOT
      for b in range(NSLOT):
        wait_g(j0 + b, b)
        fire_s(j0 + b, b)
      for b in range(NSLOT):
        wait_s(j0 + b, b)
        jn = jnp.minimum(j0 + NSLOT + b, ch_per_worker - 1)
        fire_g(jn, b)
      return 0
    lax.fori_loop(0, ngrp, grp, 0)
    for b in range(NSLOT):
      wait_g(ch_per_worker - 1, b)
    plsc.subcore_barrier()

    def wb(k, _):
      r0 = base_r + k * CHUNK
      pltpu.sync_copy(accum.at[pl.ds(r0, CHUNK)], rows[0])
      pltpu.sync_copy(rows[0], out_hbm.at[cid, pl.ds(r0, CHUNK)])
      return 0
    lax.fori_loop(0, n_row_blk, wb, 0)

  return pl.kernel(
      body,
      out_type=jax.ShapeDtypeStruct((NC, npad, d), jnp.float32),
      mesh=plsc.VectorSubcoreMesh(core_axis_name="c", subcore_axis_name="s"),
      compiler_params=pltpu.CompilerParams(use_tc_tiling_on_sc=False),
      scratch_types=(
          [pltpu.VMEM((ch_per_worker, CHUNK), jnp.int32),
           pltpu.VMEM((ch_per_worker, CHUNK), jnp.int32)]
          + [pltpu.VMEM((CHUNK, d), jnp.float32) for _ in range(NSLOT)]
          + [pltpu.VMEM_SHARED((npad, d), jnp.float32)]
          + [pltpu.SemaphoreType.DMA for _ in range(2 * NSLOT)]
      ),
  )(table, src2d, dst2d)


def _sc_degree(dst2d, npad, ch_per_worker):
  d = 16
  rows_per_sub = npad // NS
  n_row_blk = rows_per_sub // CHUNK

  def body(dst_hbm, out_hbm, idx_d, rows, accum):
    cid = lax.axis_index("c")
    sid = lax.axis_index("s")
    w = sid * NC + cid
    base_r = sid * rows_per_sub

    def zr(i, _):
      rows[i, pl.ds(0, 16)] = jnp.zeros((16,), jnp.float32)
      return 0
    lax.fori_loop(0, CHUNK, zr, 0)

    def zb(k, _):
      pltpu.sync_copy(rows, accum.at[pl.ds(base_r + k * CHUNK, CHUNK)])
      return 0
    lax.fori_loop(0, n_row_blk, zb, 0)

    def on(i, _):
      rows[i, pl.ds(0, 16)] = jnp.ones((16,), jnp.float32)
      return 0
    lax.fori_loop(0, CHUNK, on, 0)

    pltpu.sync_copy(dst_hbm.at[pl.ds(w * ch_per_worker, ch_per_worker)], idx_d)
    plsc.subcore_barrier()

    def step(j, _):
      pltpu.sync_copy(rows, accum.at[idx_d.at[j]], add=True)
      return 0
    lax.fori_loop(0, ch_per_worker, step, 0)
    plsc.subcore_barrier()

    def wb(k, _):
      r0 = base_r + k * CHUNK
      pltpu.sync_copy(accum.at[pl.ds(r0, CHUNK)], rows)
      pltpu.sync_copy(rows, out_hbm.at[cid, pl.ds(r0, CHUNK)])
      return 0
    lax.fori_loop(0, n_row_blk, wb, 0)

  return pl.kernel(
      body,
      out_type=jax.ShapeDtypeStruct((NC, npad, d), jnp.float32),
      mesh=plsc.VectorSubcoreMesh(core_axis_name="c", subcore_axis_name="s"),
      compiler_params=pltpu.CompilerParams(use_tc_tiling_on_sc=False),
      scratch_types=[
          pltpu.VMEM((ch_per_worker, CHUNK), jnp.int32),
          pltpu.VMEM((CHUNK, d), jnp.float32),
          pltpu.VMEM_SHARED((npad, d), jnp.float32),
      ],
  )(dst2d)


def _dinv_of(dp_ref):
  return lax.rsqrt(dp_ref[0, :, 0:1] + dp_ref[1, :, 0:1] + 1.0)


def _tc_layer1(deg_parts, x_pad, w1, npad, f, h):
  hh = h // 2

  def body(dp, xr, w1r, g1a, g1b):
    dinv = _dinv_of(dp)
    g1 = dinv * jnp.dot(xr[...], w1r[...], preferred_element_type=jnp.float32)
    g1a[...] = g1[:, :hh]
    g1b[...] = g1[:, hh:]
  return pl.pallas_call(
      body,
      grid=(npad // MB,),
      in_specs=[
          pl.BlockSpec((NC, MB, 16), lambda i: (0, i, 0)),
          pl.BlockSpec((MB, f), lambda i: (i, 0)),
          pl.BlockSpec((f, h), lambda i: (0, 0)),
      ],
      out_specs=[
          pl.BlockSpec((MB, hh), lambda i: (i, 0)),
          pl.BlockSpec((MB, hh), lambda i: (i, 0)),
      ],
      out_shape=[
          jax.ShapeDtypeStruct((npad, hh), jnp.float32),
          jax.ShapeDtypeStruct((npad, hh), jnp.float32),
      ],
  )(deg_parts, x_pad, w1)


def _tc_layer2(deg_parts, s1a, s1b, g1a, g1b, b1, w2, npad, h, c):
  hh = h // 2

  def body(dp, s1ar, s1br, g1ar, g1br, b1r, w2r, g2):
    dinv = _dinv_of(dp)
    ma = s1ar[0] + s1ar[1] + g1ar[...]
    mb = s1br[0] + s1br[1] + g1br[...]
    h1 = dinv * jnp.concatenate([ma, mb], axis=1) + b1r[...]
    h1 = jnp.maximum(h1, 0.0)
    g2[...] = dinv * jnp.dot(h1, w2r[...], preferred_element_type=jnp.float32)
  return pl.pallas_call(
      body,
      grid=(npad // MB,),
      in_specs=[
          pl.BlockSpec((NC, MB, 16), lambda i: (0, i, 0)),
          pl.BlockSpec((NC, MB, hh), lambda i: (0, i, 0)),
          pl.BlockSpec((NC, MB, hh), lambda i: (0, i, 0)),
          pl.BlockSpec((MB, hh), lambda i: (i, 0)),
          pl.BlockSpec((MB, hh), lambda i: (i, 0)),
          pl.BlockSpec((1, h), lambda i: (0, 0)),
          pl.BlockSpec((h, c), lambda i: (0, 0)),
      ],
      out_specs=pl.BlockSpec((MB, c), lambda i: (i, 0)),
      out_shape=jax.ShapeDtypeStruct((npad, c), jnp.float32),
  )(deg_parts, s1a, s1b, g1a, g1b, b1, w2)


def _tc_final(deg_parts, s2, g2, b2, npad, c):
  def body(dp, s2r, g2r, b2r, o):
    dinv = _dinv_of(dp)
    o[...] = dinv * (s2r[0] + s2r[1] + g2r[...]) + b2r[...]
  return pl.pallas_call(
      body,
      grid=(npad // MB,),
      in_specs=[
          pl.BlockSpec((NC, MB, 16), lambda i: (0, i, 0)),
          pl.BlockSpec((NC, MB, c), lambda i: (0, i, 0)),
          pl.BlockSpec((MB, c), lambda i: (i, 0)),
          pl.BlockSpec((1, c), lambda i: (0, 0)),
      ],
      out_specs=pl.BlockSpec((MB, c), lambda i: (i, 0)),
      out_shape=jax.ShapeDtypeStruct((npad, c), jnp.float32),
  )(deg_parts, s2, g2, b2)


def kernel(x, edge_index, W1, b1, W2, b2):
  n, f = x.shape
  h = W1.shape[1]
  c = W2.shape[1]
  e = edge_index.shape[1]

  npad = -(-(n + 1) // (NS * CHUNK)) * (NS * CHUNK)
  ch_per_worker = -(-(-(-e // (NC * NS * CHUNK))) // NSLOT) * NSLOT
  erows = ch_per_worker * NC * NS
  epad = erows * CHUNK

  src = edge_index[0]
  dst = edge_index[1]
  pad_idx = jnp.full((epad - e,), n, dtype=jnp.int32)
  src2d = jnp.concatenate([src, pad_idx]).reshape(erows, CHUNK)
  dst2d = jnp.concatenate([dst, pad_idx]).reshape(erows, CHUNK)
  x_pad = jnp.pad(x, ((0, npad - n), (0, 0)))

  deg_parts = _sc_degree(dst2d, npad, ch_per_worker)
  g1a, g1b = _tc_layer1(deg_parts, x_pad, W1, npad, f, h)
  s1a = _sc_edge_scatter(g1a, src2d, dst2d, npad, h // 2, ch_per_worker)
  s1b = _sc_edge_scatter(g1b, src2d, dst2d, npad, h // 2, ch_per_worker)
  g2 = _tc_layer2(deg_parts, s1a, s1b, g1a, g1b, b1.reshape(1, h), W2,
                  npad, h, c)
  s2 = _sc_edge_scatter(g2, src2d, dst2d, npad, c, ch_per_worker)
  out = _tc_final(deg_parts, s2, g2, b2.reshape(1, c), npad, c)
  return out[:n]


# speedup vs baseline: 1.2209x; 1.2209x over previous
import jax
import jax.numpy as jnp
from jax import lax
from jax.experimental import pallas as pl
from jax.experimental.pallas import tpu as pltpu
from jax.experimental.pallas import tpu_sc as plsc

NC = 2
NS = 16
CHUNK = 128
NSLOT = 8
MB = 256


def _sc_edge_scatter(table, src2d, dst2d, npad, d, ch_per_worker, nslot,
                     split_dst_stage=False):
  rows_per_sub = npad // NS
  n_row_blk = rows_per_sub // CHUNK
  nz = CHUNK * (d // 16)
  nphase = 2 if split_dst_stage else 1
  assert ch_per_worker % (nslot * nphase) == 0
  ch_ph = ch_per_worker // nphase
  ngrp = ch_ph // nslot

  def body(table_hbm, src_hbm, dst_hbm, out_hbm, idx_s, idx_d, *rest):
    rows = list(rest[:nslot])
    accum = rest[nslot]
    gsem = list(rest[nslot + 1:2 * nslot + 1])
    cid = lax.axis_index("c")
    sid = lax.axis_index("s")
    w = sid * NC + cid
    base_r = sid * rows_per_sub

    def zr(i, _):
      rows[0][i // (d // 16), pl.ds((i % (d // 16)) * 16, 16)] = jnp.zeros(
          (16,), jnp.float32)
      return 0
    lax.fori_loop(0, nz, zr, 0)

    def zb(k, _):
      pltpu.sync_copy(rows[0], accum.at[pl.ds(base_r + k * CHUNK, CHUNK)])
      return 0
    lax.fori_loop(0, n_row_blk, zb, 0)

    pltpu.sync_copy(src_hbm.at[pl.ds(w * ch_per_worker, ch_per_worker)], idx_s)
    pltpu.sync_copy(dst_hbm.at[pl.ds(w * ch_per_worker, ch_ph)], idx_d)
    plsc.subcore_barrier()

    def fire_g(j, b):
      pltpu.async_copy(table_hbm.at[idx_s.at[j]], rows[b], gsem[b])

    def wait_g(j, b):
      pltpu.make_async_copy(table_hbm.at[idx_s.at[j]], rows[b],
                            gsem[b]).wait()

    for b in range(nslot):
      fire_g(b, b)

    for ph in range(nphase):
      if ph:
        pltpu.sync_copy(
            dst_hbm.at[pl.ds(w * ch_per_worker + ph * ch_ph, ch_ph)], idx_d)
      j_base = ph * ch_ph

      def grp(g, _):
        j0 = j_base + g * nslot
        for b in range(nslot):
          wait_g(j0 + b, b)
          pltpu.sync_copy(rows[b], accum.at[idx_d.at[(j0 + b) - j_base]],
                          add=True)
          jn = jnp.minimum(j0 + nslot + b, ch_per_worker - 1)
          fire_g(jn, b)
        return 0
      lax.fori_loop(0, ngrp, grp, 0)
    for b in range(nslot):
      wait_g(ch_per_worker - 1, b)
    plsc.subcore_barrier()

    def wb(k, _):
      r0 = base_r + k * CHUNK
      pltpu.sync_copy(accum.at[pl.ds(r0, CHUNK)], rows[0])
      pltpu.sync_copy(rows[0], out_hbm.at[cid, pl.ds(r0, CHUNK)])
      return 0
    lax.fori_loop(0, n_row_blk, wb, 0)

  return pl.kernel(
      body,
      out_type=jax.ShapeDtypeStruct((NC, npad, d), jnp.float32),
      mesh=plsc.VectorSubcoreMesh(core_axis_name="c", subcore_axis_name="s"),
      compiler_params=pltpu.CompilerParams(use_tc_tiling_on_sc=False),
      scratch_types=(
          [pltpu.VMEM((ch_per_worker, CHUNK), jnp.int32),
           pltpu.VMEM((ch_ph, CHUNK), jnp.int32)]
          + [pltpu.VMEM((CHUNK, d), jnp.float32) for _ in range(nslot)]
          + [pltpu.VMEM_SHARED((npad, d), jnp.float32)]
          + [pltpu.SemaphoreType.DMA for _ in range(nslot)]
      ),
  )(table, src2d, dst2d)


def _sc_degree(dst2d, npad, ch_per_worker):
  d = 16
  rows_per_sub = npad // NS
  n_row_blk = rows_per_sub // CHUNK

  def body(dst_hbm, out_hbm, idx_d, rows, accum):
    cid = lax.axis_index("c")
    sid = lax.axis_index("s")
    w = sid * NC + cid
    base_r = sid * rows_per_sub

    def zr(i, _):
      rows[i, pl.ds(0, 16)] = jnp.zeros((16,), jnp.float32)
      return 0
    lax.fori_loop(0, CHUNK, zr, 0)

    def zb(k, _):
      pltpu.sync_copy(rows, accum.at[pl.ds(base_r + k * CHUNK, CHUNK)])
      return 0
    lax.fori_loop(0, n_row_blk, zb, 0)

    def on(i, _):
      rows[i, pl.ds(0, 16)] = jnp.ones((16,), jnp.float32)
      return 0
    lax.fori_loop(0, CHUNK, on, 0)

    pltpu.sync_copy(dst_hbm.at[pl.ds(w * ch_per_worker, ch_per_worker)], idx_d)
    plsc.subcore_barrier()

    def step(j, _):
      pltpu.sync_copy(rows, accum.at[idx_d.at[j]], add=True)
      return 0
    lax.fori_loop(0, ch_per_worker, step, 0)
    plsc.subcore_barrier()

    def wb(k, _):
      r0 = base_r + k * CHUNK
      pltpu.sync_copy(accum.at[pl.ds(r0, CHUNK)], rows)
      pltpu.sync_copy(rows, out_hbm.at[cid, pl.ds(r0, CHUNK)])
      return 0
    lax.fori_loop(0, n_row_blk, wb, 0)

  return pl.kernel(
      body,
      out_type=jax.ShapeDtypeStruct((NC, npad, d), jnp.float32),
      mesh=plsc.VectorSubcoreMesh(core_axis_name="c", subcore_axis_name="s"),
      compiler_params=pltpu.CompilerParams(use_tc_tiling_on_sc=False),
      scratch_types=[
          pltpu.VMEM((ch_per_worker, CHUNK), jnp.int32),
          pltpu.VMEM((CHUNK, d), jnp.float32),
          pltpu.VMEM_SHARED((npad, d), jnp.float32),
      ],
  )(dst2d)


def _dinv_of(dp_ref):
  return lax.rsqrt(dp_ref[0, :, 0:1] + dp_ref[1, :, 0:1] + 1.0)


def _tc_layer1(deg_parts, x_pad, w1, npad, f, h):
  def body(dp, xr, w1r, g1):
    dinv = _dinv_of(dp)
    g1[...] = dinv * jnp.dot(xr[...], w1r[...],
                             preferred_element_type=jnp.float32)
  return pl.pallas_call(
      body,
      grid=(npad // MB,),
      in_specs=[
          pl.BlockSpec((NC, MB, 16), lambda i: (0, i, 0)),
          pl.BlockSpec((MB, f), lambda i: (i, 0)),
          pl.BlockSpec((f, h), lambda i: (0, 0)),
      ],
      out_specs=pl.BlockSpec((MB, h), lambda i: (i, 0)),
      out_shape=jax.ShapeDtypeStruct((npad, h), jnp.float32),
  )(deg_parts, x_pad, w1)


def _tc_layer2(deg_parts, s1, g1, b1, w2, npad, h, c):
  def body(dp, s1r, g1r, b1r, w2r, g2):
    dinv = _dinv_of(dp)
    h1 = dinv * (s1r[0] + s1r[1] + g1r[...]) + b1r[...]
    h1 = jnp.maximum(h1, 0.0)
    g2[...] = dinv * jnp.dot(h1, w2r[...], preferred_element_type=jnp.float32)
  return pl.pallas_call(
      body,
      grid=(npad // MB,),
      in_specs=[
          pl.BlockSpec((NC, MB, 16), lambda i: (0, i, 0)),
          pl.BlockSpec((NC, MB, h), lambda i: (0, i, 0)),
          pl.BlockSpec((MB, h), lambda i: (i, 0)),
          pl.BlockSpec((1, h), lambda i: (0, 0)),
          pl.BlockSpec((h, c), lambda i: (0, 0)),
      ],
      out_specs=pl.BlockSpec((MB, c), lambda i: (i, 0)),
      out_shape=jax.ShapeDtypeStruct((npad, c), jnp.float32),
  )(deg_parts, s1, g1, b1, w2)


def _tc_final(deg_parts, s2, g2, b2, npad, c):
  def body(dp, s2r, g2r, b2r, o):
    dinv = _dinv_of(dp)
    o[...] = dinv * (s2r[0] + s2r[1] + g2r[...]) + b2r[...]
  return pl.pallas_call(
      body,
      grid=(npad // MB,),
      in_specs=[
          pl.BlockSpec((NC, MB, 16), lambda i: (0, i, 0)),
          pl.BlockSpec((NC, MB, c), lambda i: (0, i, 0)),
          pl.BlockSpec((MB, c), lambda i: (i, 0)),
          pl.BlockSpec((1, c), lambda i: (0, 0)),
      ],
      out_specs=pl.BlockSpec((MB, c), lambda i: (i, 0)),
      out_shape=jax.ShapeDtypeStruct((npad, c), jnp.float32),
  )(deg_parts, s2, g2, b2)


def kernel(x, edge_index, W1, b1, W2, b2):
  n, f = x.shape
  h = W1.shape[1]
  c = W2.shape[1]
  e = edge_index.shape[1]

  npad = -(-(n + 1) // (NS * CHUNK)) * (NS * CHUNK)
  ch_per_worker = -(-(-(-e // (NC * NS * CHUNK))) // NSLOT) * NSLOT
  erows = ch_per_worker * NC * NS
  epad = erows * CHUNK

  src = edge_index[0]
  dst = edge_index[1]
  pad_idx = jnp.full((epad - e,), n, dtype=jnp.int32)
  src2d = jnp.concatenate([src, pad_idx]).reshape(erows, CHUNK)
  dst2d = jnp.concatenate([dst, pad_idx]).reshape(erows, CHUNK)
  x_pad = jnp.pad(x, ((0, npad - n), (0, 0)))

  deg_parts = _sc_degree(dst2d, npad, ch_per_worker)
  g1 = _tc_layer1(deg_parts, x_pad, W1, npad, f, h)
  s1 = _sc_edge_scatter(g1, src2d, dst2d, npad, h, ch_per_worker, 2,
                        split_dst_stage=True)
  g2 = _tc_layer2(deg_parts, s1, g1, b1.reshape(1, h), W2, npad, h, c)
  s2 = _sc_edge_scatter(g2, src2d, dst2d, npad, c, ch_per_worker, 8)
  out = _tc_final(deg_parts, s2, g2, b2.reshape(1, c), npad, c)
  return out[:n]


# speedup vs baseline: 1.6369x; 1.3408x over previous
import jax
import jax.numpy as jnp
from jax import lax
from jax.experimental import pallas as pl
from jax.experimental.pallas import tpu as pltpu
from jax.experimental.pallas import tpu_sc as plsc

NC = 2
NS = 16
CHUNK = 128
CH_ALIGN = 160
MB = 256


def _zero_accum_slice(rows0, accum, base_r, n_row_blk, nz, d):
  def zr(i, _):
    rows0[i // (d // 16), pl.ds((i % (d // 16)) * 16, 16)] = jnp.zeros(
        (16,), jnp.float32)
    return 0
  lax.fori_loop(0, nz, zr, 0)

  def zb(k, _):
    pltpu.sync_copy(rows0, accum.at[pl.ds(base_r + k * CHUNK, CHUNK)])
    return 0
  lax.fori_loop(0, n_row_blk, zb, 0)


def _writeback(rows0, accum, out_ref, base_r, n_row_blk):
  def wb(k, _):
    r0 = base_r + k * CHUNK
    pltpu.sync_copy(accum.at[pl.ds(r0, CHUNK)], rows0)
    pltpu.sync_copy(rows0, out_ref.at[pl.ds(r0, CHUNK)])
    return 0
  lax.fori_loop(0, n_row_blk, wb, 0)


def _sc_edge_scatter_cols(table2, src2d, dst2d, npad, dh, ch_sub, nslot):
  rows_per_sub = npad // NS
  n_row_blk = rows_per_sub // CHUNK
  nz = CHUNK * (dh // 16)
  assert ch_sub % nslot == 0
  ngrp = ch_sub // nslot

  def body(table_hbm, src_hbm, dst_hbm, out_hbm, idx_s, idx_d, *rest):
    rows = list(rest[:nslot])
    accum = rest[nslot]
    gsem = list(rest[nslot + 1:2 * nslot + 1])
    cid = lax.axis_index("c")
    sid = lax.axis_index("s")
    base_r = sid * rows_per_sub
    base_c = sid * ch_sub

    _zero_accum_slice(rows[0], accum, base_r, n_row_blk, nz, dh)

    pltpu.sync_copy(src_hbm.at[pl.ds(base_c, ch_sub)], idx_s)
    pltpu.sync_copy(dst_hbm.at[pl.ds(base_c, ch_sub)], idx_d)
    plsc.subcore_barrier()

    def fire_g(j, b):
      pltpu.async_copy(table_hbm.at[cid].at[idx_s.at[j]], rows[b], gsem[b])

    def wait_g(j, b):
      pltpu.make_async_copy(table_hbm.at[cid].at[idx_s.at[j]], rows[b],
                            gsem[b]).wait()

    for b in range(nslot):
      fire_g(b, b)

    def grp(g, _):
      j0 = g * nslot
      for b in range(nslot):
        wait_g(j0 + b, b)
        pltpu.sync_copy(rows[b], accum.at[idx_d.at[j0 + b]], add=True)
        jn = jnp.minimum(j0 + nslot + b, ch_sub - 1)
        fire_g(jn, b)
      return 0
    lax.fori_loop(0, ngrp, grp, 0)
    for b in range(nslot):
      wait_g(ch_sub - 1, b)
    plsc.subcore_barrier()

    _writeback(rows[0], accum, out_hbm.at[cid], base_r, n_row_blk)

  return pl.kernel(
      body,
      out_type=jax.ShapeDtypeStruct((NC, npad, dh), jnp.float32),
      mesh=plsc.VectorSubcoreMesh(core_axis_name="c", subcore_axis_name="s"),
      compiler_params=pltpu.CompilerParams(use_tc_tiling_on_sc=False),
      scratch_types=(
          [pltpu.VMEM((ch_sub, CHUNK), jnp.int32),
           pltpu.VMEM((ch_sub, CHUNK), jnp.int32)]
          + [pltpu.VMEM((CHUNK, dh), jnp.float32) for _ in range(nslot)]
          + [pltpu.VMEM_SHARED((npad, dh), jnp.float32)]
          + [pltpu.SemaphoreType.DMA for _ in range(nslot)]
      ),
  )(table2, src2d, dst2d)


def _sc_edge_scatter(table, src2d, dst2d, npad, d, ch_sub, nslot):
  rows_per_sub = npad // NS
  n_row_blk = rows_per_sub // CHUNK
  nz = CHUNK * (d // 16)
  chw = ch_sub // NC
  assert chw % nslot == 0
  ngrp = chw // nslot

  def body(table_hbm, src_hbm, dst_hbm, out_hbm, idx_s, idx_d, *rest):
    rows = list(rest[:nslot])
    accum = rest[nslot]
    gsem = list(rest[nslot + 1:2 * nslot + 1])
    cid = lax.axis_index("c")
    sid = lax.axis_index("s")
    base_r = sid * rows_per_sub
    base_c = sid * ch_sub + cid * chw

    _zero_accum_slice(rows[0], accum, base_r, n_row_blk, nz, d)

    pltpu.sync_copy(src_hbm.at[pl.ds(base_c, chw)], idx_s)
    pltpu.sync_copy(dst_hbm.at[pl.ds(base_c, chw)], idx_d)
    plsc.subcore_barrier()

    def fire_g(j, b):
      pltpu.async_copy(table_hbm.at[idx_s.at[j]], rows[b], gsem[b])

    def wait_g(j, b):
      pltpu.make_async_copy(table_hbm.at[idx_s.at[j]], rows[b],
                            gsem[b]).wait()

    for b in range(nslot):
      fire_g(b, b)

    def grp(g, _):
      j0 = g * nslot
      for b in range(nslot):
        wait_g(j0 + b, b)
        pltpu.sync_copy(rows[b], accum.at[idx_d.at[j0 + b]], add=True)
        jn = jnp.minimum(j0 + nslot + b, chw - 1)
        fire_g(jn, b)
      return 0
    lax.fori_loop(0, ngrp, grp, 0)
    for b in range(nslot):
      wait_g(chw - 1, b)
    plsc.subcore_barrier()

    _writeback(rows[0], accum, out_hbm.at[cid], base_r, n_row_blk)

  return pl.kernel(
      body,
      out_type=jax.ShapeDtypeStruct((NC, npad, d), jnp.float32),
      mesh=plsc.VectorSubcoreMesh(core_axis_name="c", subcore_axis_name="s"),
      compiler_params=pltpu.CompilerParams(use_tc_tiling_on_sc=False),
      scratch_types=(
          [pltpu.VMEM((chw, CHUNK), jnp.int32),
           pltpu.VMEM((chw, CHUNK), jnp.int32)]
          + [pltpu.VMEM((CHUNK, d), jnp.float32) for _ in range(nslot)]
          + [pltpu.VMEM_SHARED((npad, d), jnp.float32)]
          + [pltpu.SemaphoreType.DMA for _ in range(nslot)]
      ),
  )(table, src2d, dst2d)


def _sc_degree(dst2d, npad, ch_sub):
  d = 16
  rows_per_sub = npad // NS
  n_row_blk = rows_per_sub // CHUNK
  chw = ch_sub // NC

  def body(dst_hbm, out_hbm, idx_d, rows, accum):
    cid = lax.axis_index("c")
    sid = lax.axis_index("s")
    base_r = sid * rows_per_sub
    base_c = sid * ch_sub + cid * chw

    _zero_accum_slice(rows, accum, base_r, n_row_blk, CHUNK, d)

    def on(i, _):
      rows[i, pl.ds(0, 16)] = jnp.ones((16,), jnp.float32)
      return 0
    lax.fori_loop(0, CHUNK, on, 0)

    pltpu.sync_copy(dst_hbm.at[pl.ds(base_c, chw)], idx_d)
    plsc.subcore_barrier()

    def step(j, _):
      pltpu.sync_copy(rows, accum.at[idx_d.at[j]], add=True)
      return 0
    lax.fori_loop(0, chw, step, 0)
    plsc.subcore_barrier()

    _writeback(rows, accum, out_hbm.at[cid], base_r, n_row_blk)

  return pl.kernel(
      body,
      out_type=jax.ShapeDtypeStruct((NC, npad, d), jnp.float32),
      mesh=plsc.VectorSubcoreMesh(core_axis_name="c", subcore_axis_name="s"),
      compiler_params=pltpu.CompilerParams(use_tc_tiling_on_sc=False),
      scratch_types=[
          pltpu.VMEM((chw, CHUNK), jnp.int32),
          pltpu.VMEM((CHUNK, d), jnp.float32),
          pltpu.VMEM_SHARED((npad, d), jnp.float32),
      ],
  )(dst2d)


def _dinv_of(dp_ref):
  return lax.rsqrt(dp_ref[0, :, 0:1] + dp_ref[1, :, 0:1] + 1.0)


def _tc_layer1(deg_parts, x_pad, w1, npad, f, h):
  hh = h // 2

  def body(dp, xr, w1r, g1o):
    dinv = _dinv_of(dp)
    g1 = dinv * jnp.dot(xr[...], w1r[...], preferred_element_type=jnp.float32)
    g1o[0] = g1[:, :hh]
    g1o[1] = g1[:, hh:]
  return pl.pallas_call(
      body,
      grid=(npad // MB,),
      in_specs=[
          pl.BlockSpec((NC, MB, 16), lambda i: (0, i, 0)),
          pl.BlockSpec((MB, f), lambda i: (i, 0)),
          pl.BlockSpec((f, h), lambda i: (0, 0)),
      ],
      out_specs=pl.BlockSpec((NC, MB, hh), lambda i: (0, i, 0)),
      out_shape=jax.ShapeDtypeStruct((NC, npad, hh), jnp.float32),
  )(deg_parts, x_pad, w1)


def _tc_layer2(deg_parts, s1, g1, b1, w2, npad, h, c):
  hh = h // 2

  def body(dp, s1r, g1r, b1r, w2r, g2):
    dinv = _dinv_of(dp)
    m = jnp.concatenate([s1r[0] + g1r[0], s1r[1] + g1r[1]], axis=1)
    h1 = jnp.maximum(dinv * m + b1r[...], 0.0)
    g2[...] = dinv * jnp.dot(h1, w2r[...], preferred_element_type=jnp.float32)
  return pl.pallas_call(
      body,
      grid=(npad // MB,),
      in_specs=[
          pl.BlockSpec((NC, MB, 16), lambda i: (0, i, 0)),
          pl.BlockSpec((NC, MB, hh), lambda i: (0, i, 0)),
          pl.BlockSpec((NC, MB, hh), lambda i: (0, i, 0)),
          pl.BlockSpec((1, h), lambda i: (0, 0)),
          pl.BlockSpec((h, c), lambda i: (0, 0)),
      ],
      out_specs=pl.BlockSpec((MB, c), lambda i: (i, 0)),
      out_shape=jax.ShapeDtypeStruct((npad, c), jnp.float32),
  )(deg_parts, s1, g1, b1, w2)


def _tc_final(deg_parts, s2, g2, b2, npad, c):
  def body(dp, s2r, g2r, b2r, o):
    dinv = _dinv_of(dp)
    o[...] = dinv * (s2r[0] + s2r[1] + g2r[...]) + b2r[...]
  return pl.pallas_call(
      body,
      grid=(npad // MB,),
      in_specs=[
          pl.BlockSpec((NC, MB, 16), lambda i: (0, i, 0)),
          pl.BlockSpec((NC, MB, c), lambda i: (0, i, 0)),
          pl.BlockSpec((MB, c), lambda i: (i, 0)),
          pl.BlockSpec((1, c), lambda i: (0, 0)),
      ],
      out_specs=pl.BlockSpec((MB, c), lambda i: (i, 0)),
      out_shape=jax.ShapeDtypeStruct((npad, c), jnp.float32),
  )(deg_parts, s2, g2, b2)


def kernel(x, edge_index, W1, b1, W2, b2):
  n, f = x.shape
  h = W1.shape[1]
  c = W2.shape[1]
  e = edge_index.shape[1]

  npad = -(-(n + 1) // (NS * CHUNK)) * (NS * CHUNK)
  ch_min = -(-e // (NS * CHUNK))
  ch_sub = -(-ch_min // CH_ALIGN) * CH_ALIGN
  erows = ch_sub * NS
  epad = erows * CHUNK

  src = edge_index[0]
  dst = edge_index[1]
  pad_idx = jnp.full((epad - e,), n, dtype=jnp.int32)
  src2d = jnp.concatenate([src, pad_idx]).reshape(erows, CHUNK)
  dst2d = jnp.concatenate([dst, pad_idx]).reshape(erows, CHUNK)
  x_pad = jnp.pad(x, ((0, npad - n), (0, 0)))

  deg_parts = _sc_degree(dst2d, npad, ch_sub)
  g1 = _tc_layer1(deg_parts, x_pad, W1, npad, f, h)
  s1 = _sc_edge_scatter_cols(g1, src2d, dst2d, npad, h // 2, ch_sub, 5)
  g2 = _tc_layer2(deg_parts, s1, g1, b1.reshape(1, h), W2, npad, h, c)
  s2 = _sc_edge_scatter(g2, src2d, dst2d, npad, c, ch_sub, 16)
  out = _tc_final(deg_parts, s2, g2, b2.reshape(1, c), npad, c)
  return out[:n]
